# Initial kernel scaffold; baseline (speedup 1.0000x reference)
import jax
import jax.numpy as jnp
from jax.experimental import pallas as pl


def kernel(x, edge_index, params):
    raise NotImplementedError("write your pallas kernel here")


# speedup vs baseline: 58.1133x; 58.1133x over previous
import functools
from typing import Any

import jax
import jax.numpy as jnp
import numpy as np
from jax import lax
from jax.experimental import pallas as pl
from jax.experimental.pallas import tpu as pltpu
from jax.experimental.pallas import tpu_sc as plsc

N_NODES = 10000
NP = 10240
R = 512
H = 50
NW = 32
NS_W = NP // NW
BLK = 128
_INTERP = False
_USE_SC = True


def _tcA_body(apply_bn, nch, w, *refs):
    if apply_bn:
        (hraw_ref, bn_ref, gamma_ref, beta_ref, Amat_ref, Bmat_ref, bpre_ref,
         bt_ref, c_ref, h_ref) = refs
        hraw = hraw_ref[...]
        F = hraw.shape[-1]
        bs = bn_ref[...]
        mu = bs[0:1, :F] / float(N_NODES)
        var = bs[1:2, :F] / float(N_NODES) - mu * mu
        h = jax.nn.relu(gamma_ref[...] * (hraw - mu) * lax.rsqrt(var + 1e-5)
                        + beta_ref[...])
    else:
        (hraw_ref, Amat_ref, Bmat_ref, bpre_ref, bt_ref, c_ref, h_ref) = refs
        h = hraw_ref[...]
    b = jnp.dot(h, Bmat_ref[...], preferred_element_type=jnp.float32)
    c = jnp.dot(h, Amat_ref[...], preferred_element_type=jnp.float32) \
        + bpre_ref[...]
    for ci in range(nch):
        bt_ref[ci] = b[:, ci * w:(ci + 1) * w]
        c_ref[ci] = c[:, ci * w:(ci + 1) * w]
    h_ref[...] = h


def _tcA(i_layer, apply_bn, nch, w, F, hraw, bn, gamma, beta, Amat, Bmat, bpre):
    grid = (NP // R,)
    full = lambda shape: pl.BlockSpec(shape, lambda i: (0,) * len(shape))
    in_specs = [pl.BlockSpec((R, F), lambda i: (i, 0))]
    args = [hraw]
    if apply_bn:
        in_specs += [full((8, 64)), full((1, F)), full((1, F))]
        args += [bn, gamma.reshape(1, F), beta.reshape(1, F)]
    in_specs += [full((F, nch * w)), full((F, nch * w)), full((1, nch * w))]
    args += [Amat, Bmat, bpre]
    out_specs = [
        pl.BlockSpec((nch, R, w), lambda i: (0, i, 0)),
        pl.BlockSpec((nch, R, w), lambda i: (0, i, 0)),
        pl.BlockSpec((R, F), lambda i: (i, 0)),
    ]
    out_shape = [
        jax.ShapeDtypeStruct((nch, NP, w), jnp.float32),
        jax.ShapeDtypeStruct((nch, NP, w), jnp.float32),
        jax.ShapeDtypeStruct((NP, F), jnp.float32),
    ]
    return pl.pallas_call(
        functools.partial(_tcA_body, apply_bn, nch, w),
        grid=grid, in_specs=in_specs, out_specs=out_specs,
        out_shape=out_shape, interpret=_INTERP,
        name=f"pna_pre_{i_layer}",
    )(*args)


def _tcB_body(nch, w, has_gru, *refs):
    if has_gru:
        (h_ref, c_ref, s1_ref, s2_ref, mn_ref, mx_ref, cnt_ref, avg_ref,
         Wx_ref, BD1_ref, BD2_ref, BD3_ref, bpost_ref, Wlin_ref, blin_ref,
         Wih_ref, Whh_ref, bih_ref, bhh_ref,
         hout_ref, bn_ref) = refs
    else:
        (h_ref, c_ref, s1_ref, s2_ref, mn_ref, mx_ref, cnt_ref, avg_ref,
         Wx_ref, BD1_ref, BD2_ref, BD3_ref, bpost_ref, Wlin_ref, blin_ref,
         yout_ref) = refs
    h = h_ref[...]
    counts = cnt_ref[...]
    avg_log = avg_ref[0, 0]
    deg = jnp.maximum(counts, 1.0)
    logdeg = jnp.log(deg + 1.0)
    sc1 = logdeg / avg_log
    sc2 = avg_log / logdeg
    cnt3 = counts[None]
    deg3 = deg[None]
    s1 = s1_ref[...]
    s2 = s2_ref[...]
    mean = s1 / deg3
    sq = s2 / deg3
    std = jnp.sqrt(jax.nn.relu(sq - mean * mean) + 1e-5)
    has3 = cnt3 > 0.0
    mn = jnp.where(has3, mn_ref[...], 0.0)
    mx = jnp.where(has3, mx_ref[...], 0.0)
    stats = (mean, mn, mx, std)
    oc = Wx_ref.shape[-1]
    A1 = jnp.zeros((h.shape[0], oc), jnp.float32)
    A2 = jnp.zeros((h.shape[0], oc), jnp.float32)
    A3 = jnp.zeros((h.shape[0], oc), jnp.float32)
    for si in range(4):
        for ci in range(nch):
            g = stats[si][ci]
            A1 += jnp.dot(g, BD1_ref[si, ci], preferred_element_type=jnp.float32)
            A2 += jnp.dot(g * sc1, BD2_ref[si, ci],
                          preferred_element_type=jnp.float32)
            A3 += jnp.dot(g * sc2, BD3_ref[si, ci],
                          preferred_element_type=jnp.float32)
    y0 = (jnp.dot(h, Wx_ref[...], preferred_element_type=jnp.float32)
          + A1 + A2 + A3 + bpost_ref[...])
    y = jnp.dot(y0, Wlin_ref[...], preferred_element_type=jnp.float32) \
        + blin_ref[...]
    if not has_gru:
        yout_ref[...] = y
        return
    gi = jnp.dot(h, Wih_ref[...], preferred_element_type=jnp.float32) \
        + bih_ref[...]
    gh = jnp.dot(y, Whh_ref[...], preferred_element_type=jnp.float32) \
        + bhh_ref[...]
    i_r, i_z, i_n = gi[:, :H], gi[:, H:2 * H], gi[:, 2 * H:]
    h_r, h_z, h_n = gh[:, :H], gh[:, H:2 * H], gh[:, 2 * H:]
    r = jax.nn.sigmoid(i_r + h_r)
    z = jax.nn.sigmoid(i_z + h_z)
    nn_ = jnp.tanh(i_n + r * h_n)
    hn = (1.0 - z) * nn_ + z * y
    hout_ref[...] = hn
    pid = pl.program_id(0)
    rowmask = (lax.broadcasted_iota(jnp.int32, (hn.shape[0], 1), 0)
               + pid * hn.shape[0]) < N_NODES
    hm = jnp.where(rowmask, hn, 0.0)

    @pl.when(pid == 0)
    def _():
        bn_ref[...] = jnp.zeros_like(bn_ref)
    bn_ref[0:1, :H] += jnp.sum(hm, axis=0, keepdims=True)
    bn_ref[1:2, :H] += jnp.sum(hm * hm, axis=0, keepdims=True)


def _tcB(i_layer, nch, w, F, oc, has_gru, h, cch, s1, s2, mn, mx, counts, avg,
         wts):
    grid = (NP // R,)
    full = lambda shape: pl.BlockSpec(shape, lambda i: (0,) * len(shape))
    ch_spec = pl.BlockSpec((nch, R, w), lambda i: (0, i, 0))
    in_specs = [pl.BlockSpec((R, F), lambda i: (i, 0)),
                ch_spec, ch_spec, ch_spec, ch_spec, ch_spec,
                pl.BlockSpec((R, 1), lambda i: (i, 0)),
                pl.BlockSpec(memory_space=pltpu.SMEM)]
    args = [h, cch, s1, s2, mn, mx, counts, avg]
    tfo = wts['Wx'].shape[-1]
    in_specs += [full((F, tfo)), full((4, nch, w, tfo)), full((4, nch, w, tfo)),
                 full((4, nch, w, tfo)), full((1, tfo)), full((tfo, oc)),
                 full((1, oc))]
    args += [wts['Wx'], wts['BD1'], wts['BD2'], wts['BD3'], wts['bpost'],
             wts['Wlin'], wts['blin']]
    if has_gru:
        in_specs += [full((F, 3 * H)), full((H, 3 * H)), full((1, 3 * H)),
                     full((1, 3 * H))]
        args += [wts['Wih'], wts['Whh'], wts['bih'], wts['bhh']]
        out_specs = [pl.BlockSpec((R, H), lambda i: (i, 0)), full((8, 64))]
        out_shape = [jax.ShapeDtypeStruct((NP, H), jnp.float32),
                     jax.ShapeDtypeStruct((8, 64), jnp.float32)]
    else:
        out_specs = [pl.BlockSpec((R, oc), lambda i: (i, 0))]
        out_shape = [jax.ShapeDtypeStruct((NP, oc), jnp.float32)]
    return pl.pallas_call(
        functools.partial(_tcB_body, nch, w, has_gru),
        grid=grid, in_specs=in_specs, out_specs=out_specs,
        out_shape=out_shape, interpret=_INTERP,
        name=f"pna_post_{i_layer}",
    )(*args)


def _avg_body(cnt_ref, out_ref):
    c = cnt_ref[...]
    idx = (lax.broadcasted_iota(jnp.int32, c.shape, 0) * 128
           + lax.broadcasted_iota(jnp.int32, c.shape, 1))
    v = jnp.where(idx < N_NODES, jnp.log(c + 1.0), 0.0)
    out_ref[0, 0] = jnp.sum(v) / float(N_NODES)


def _avg_log(counts):
    return pl.pallas_call(
        _avg_body,
        in_specs=[pl.BlockSpec((80, 128), lambda: (0, 0))],
        out_specs=pl.BlockSpec(memory_space=pltpu.SMEM),
        out_shape=jax.ShapeDtypeStruct((1, 1), jnp.float32),
        interpret=_INTERP, name="pna_avglog",
    )(counts.reshape(80, 128))


def _segreduce(bt, cch, ssrc, sdst, counts_i, offsets):
    nch = bt.shape[0]
    rows = bt[:, ssrc, :] + cch[:, sdst, :]
    S1 = jax.ops.segment_sum(rows.transpose(1, 0, 2), sdst, num_segments=NP)
    S2 = jax.ops.segment_sum((rows * rows).transpose(1, 0, 2), sdst,
                             num_segments=NP)
    Mn = jax.ops.segment_min(rows.transpose(1, 0, 2), sdst, num_segments=NP)
    Mx = jax.ops.segment_max(rows.transpose(1, 0, 2), sdst, num_segments=NP)
    z = jnp.zeros((NP, nch, bt.shape[2]), jnp.float32)
    has = (counts_i > 0)[:, None, None]
    Mn = jnp.where(has, Mn, 0.0)
    Mx = jnp.where(has, Mx, 0.0)
    return (S1.transpose(1, 0, 2), S2.transpose(1, 0, 2),
            Mn.transpose(1, 0, 2), Mx.transpose(1, 0, 2))


def _sc_body(nch, w, nv, *refs):
    (bt_ref, c_hbm, ssrc_ref, pars_ref, segc_hbm, segn_hbm,
     S1_ref, S2_ref, Mn_ref, Mx_ref,
     idx_ref, idxo_ref, rows_ref, stg_ref, cvm_ref, segc_ref, segn_ref,
     pvm_ref, isem, gsem) = refs
    wid = lax.axis_index("c") * 16 + lax.axis_index("s")
    pltpu.sync_copy(pars_ref.at[wid], pvm_ref)
    pltpu.sync_copy(segc_hbm.at[wid], segc_ref)
    pltpu.sync_copy(segn_hbm.at[wid], segn_ref)
    pv = pvm_ref[...]
    o0 = pv[0]
    nE = pv[1]
    nseg = pv[2]
    k0 = lax.rem(o0, jnp.int32(8))
    o0f = o0 - k0
    nblk = (k0 + nE + BLK - 1) // BLK

    zero = jnp.zeros((16,), jnp.float32)
    pinf = jnp.full((16,), jnp.inf, jnp.float32)
    ninf = jnp.full((16,), -jnp.inf, jnp.float32)
    idents = ([zero] * nv) + ([zero] * nv) + ([pinf] * nv) + ([ninf] * nv)
    outs = (S1_ref, S2_ref, Mn_ref, Mx_ref)

    for c in range(nch):
        pltpu.sync_copy(
            c_hbm.at[pl.ds(pl.multiple_of(jnp.int32(c * NP) + wid * NS_W, 8),
                           NS_W)], cvm_ref)
        def zbody(ln, carry):
            for v in range(nv):
                stg_ref[0, ln, pl.ds(v * 16, 16)] = zero
                stg_ref[1, ln, pl.ds(v * 16, 16)] = zero
            return carry
        lax.fori_loop(0, NS_W, zbody, jnp.int32(0))

        def idx_copy(j):
            sl = lax.rem(j, 2)
            return pltpu.make_async_copy(
                ssrc_ref.at[pl.ds(pl.multiple_of(o0f + j * BLK, 8), BLK)],
                idx_ref.at[sl], isem.at[sl])

        def gather_copy(j):
            sl = lax.rem(j, 2)
            return pltpu.make_async_copy(
                bt_ref.at[idxo_ref.at[sl]], rows_ref.at[sl], gsem.at[sl])

        def start_gather(j):
            sl = lax.rem(j, 2)
            for v in range(BLK // 16):
                idxo_ref[sl, pl.ds(v * 16, 16)] = (
                    idx_ref[sl, pl.ds(v * 16, 16)] + jnp.int32(c * NP))
            gather_copy(j).start()

        @pl.when(nE > 0)
        def _():
            idx_copy(jnp.int32(0)).start()
            idx_copy(jnp.int32(0)).wait()
            start_gather(jnp.int32(0))

            @pl.when(nblk > 1)
            def _():
                idx_copy(jnp.int32(1)).start()

        def gbody(g, car):
            ep, tp0 = car
            cnt = segc_ref[pl.ds(g, 16)][0]
            node = segn_ref[pl.ds(g, 16)][0]
            cvs = [cvm_ref[node, pl.ds(v * 16, 16)] for v in range(nv)]

            def ebody(k, a):
                tp = a[0]
                e = ep + k

                @pl.when(e == tp)
                def _():
                    b = e // BLK
                    gather_copy(b).wait()

                    @pl.when(b + 1 < nblk)
                    def _():
                        idx_copy(b + 1).wait()
                        start_gather(b + 1)

                    @pl.when(b + 2 < nblk)
                    def _():
                        idx_copy(b + 2).start()

                tp = jnp.where(e == tp, (e // BLK + 1) * BLK, tp)
                sl = lax.rem(e // BLK, 2)
                r = lax.rem(e, BLK)
                vs = [rows_ref[sl, r, pl.ds(v * 16, 16)] + cvs[v]
                      for v in range(nv)]
                acc = a[1:]
                out = [acc[v] + vs[v] for v in range(nv)]
                out += [acc[nv + v] + vs[v] * vs[v] for v in range(nv)]
                out += [jnp.minimum(acc[2 * nv + v], vs[v]) for v in range(nv)]
                out += [jnp.maximum(acc[3 * nv + v], vs[v]) for v in range(nv)]
                return (tp,) + tuple(out)

            res = lax.fori_loop(0, cnt, ebody, (tp0,) + tuple(idents))
            for si in range(4):
                for v in range(nv):
                    stg_ref[si, node, pl.ds(v * 16, 16)] = res[1 + si * nv + v]
            return (ep + cnt, res[0])

        lax.fori_loop(0, nseg, gbody, (k0, k0))

        for si in range(4):
            pltpu.sync_copy(stg_ref.at[si],
                            outs[si].at[c, pl.ds(wid * NS_W, NS_W)])


def _sc_segreduce(bt, cch, ssrc_p, pars, segc, segn, nch, w):
    nv = w // 16
    nseg = segc.shape[1]
    mesh = plsc.VectorSubcoreMesh(core_axis_name="c", subcore_axis_name="s")
    out_t = [jax.ShapeDtypeStruct((nch, NP, w), jnp.float32)
             for _ in range(4)]
    scratch = [
        pltpu.VMEM((2, BLK), jnp.int32),
        pltpu.VMEM((2, BLK), jnp.int32),
        pltpu.VMEM((2, BLK, w), jnp.float32),
        pltpu.VMEM((4, NS_W, w), jnp.float32),
        pltpu.VMEM((NS_W, w), jnp.float32),
        pltpu.VMEM((nseg,), jnp.int32),
        pltpu.VMEM((nseg,), jnp.int32),
        pltpu.VMEM((16,), jnp.int32),
        pltpu.SemaphoreType.DMA((2,)),
        pltpu.SemaphoreType.DMA((2,)),
    ]
    f = pl.kernel(functools.partial(_sc_body, nch, w, nv),
                  out_type=out_t, mesh=mesh, scratch_types=scratch,
                  compiler_params=pltpu.CompilerParams(
                      use_tc_tiling_on_sc=False),
                  name=f"pna_segreduce_{nch}x{w}")
    return f(bt.reshape(nch * NP, w), cch.reshape(nch * NP, w),
             ssrc_p, pars, segc, segn)


def _prep_conv(p, F, T, nch, w):
    TF = T * F
    TFp = nch * w
    F_out = p['Wpost'].shape[-1]
    tfo = T * F_out
    A = p['Wpre'][:, :F, :].transpose(1, 0, 2).reshape(F, TF)
    B = p['Wpre'][:, F:, :].transpose(1, 0, 2).reshape(F, TF)
    pad = TFp - TF
    Amat = jnp.pad(A, ((0, 0), (0, pad)))
    Bmat = jnp.pad(B, ((0, 0), (0, pad)))
    bpre = jnp.pad(p['bpre'].reshape(1, TF), ((0, 0), (0, pad)))
    Wx = p['Wpost'][:, :F, :].transpose(1, 0, 2).reshape(F, tfo)
    eye = jnp.eye(T, dtype=jnp.float32)
    out = {'Amat': Amat, 'Bmat': Bmat, 'bpre': bpre, 'Wx': Wx}
    for j, name in ((1, 'BD1'), (2, 'BD2'), (3, 'BD3')):
        Wj = p['Wpost'][:, F + (j - 1) * 4 * F: F + j * 4 * F, :]
        Wj = Wj.reshape(T, 4, F, F_out)
        M = jnp.einsum('tsok,tu->stouk', Wj, eye).reshape(4, TF, tfo)
        M = jnp.pad(M, ((0, 0), (0, pad), (0, 0)))
        out[name] = M.reshape(4, nch, w, tfo)
    out['bpost'] = p['bpost'].reshape(1, tfo)
    out['Wlin'] = p['Wlin']
    out['blin'] = p['blin'].reshape(1, -1)
    return out


def kernel(x, edge_index, params):
    N = x.shape[0]
    assert N == N_NODES
    src = edge_index[0]
    dst = edge_index[1]
    perm = jnp.argsort(dst)
    ssrc = src[perm].astype(jnp.int32)
    sdst = dst[perm].astype(jnp.int32)
    offsets = jnp.searchsorted(sdst, jnp.arange(NP + 1, dtype=jnp.int32)
                               ).astype(jnp.int32)
    counts_i = offsets[1:] - offsets[:-1]
    counts_f = counts_i.astype(jnp.float32)
    counts_col = counts_f.reshape(NP, 1)
    avg = _avg_log(counts_f)

    ssrc_p = jnp.pad(ssrc, (0, 256))
    warange = jnp.arange(NW, dtype=jnp.int32)
    o0s = offsets[warange * NS_W]
    nEs = offsets[(warange + 1) * NS_W] - o0s
    nsegs = jnp.sum((counts_i.reshape(NW, NS_W) > 0), axis=1,
                    dtype=jnp.int32)
    pars = (jnp.zeros((NW, 16), jnp.int32)
            .at[:, 0].set(o0s).at[:, 1].set(nEs).at[:, 2].set(nsegs))
    counts2 = counts_i.reshape(NW, NS_W)
    order = jnp.argsort(counts2 == 0, axis=1, stable=True).astype(jnp.int32)
    segc = jnp.take_along_axis(counts2, order, axis=1)
    segc = jnp.where(segc == 0, jnp.int32(1 << 30), segc)
    segc = jnp.pad(segc, ((0, 0), (0, 32)), constant_values=1 << 30)
    segn = jnp.pad(order, ((0, 0), (0, 32)), constant_values=0)

    def seg(bt, cch, nch, w):
        if _USE_SC:
            return _sc_segreduce(bt, cch, ssrc_p, pars, segc, segn, nch, w)
        return _segreduce(bt, cch, ssrc, sdst, counts_i, offsets)

    layers = params['layers']
    h = jnp.pad(x, ((0, NP - N), (0, 0)))
    bn = None
    for i, lp in enumerate(layers):
        F = 2 if i == 0 else H
        T = 1 if i == 0 else 5
        nch, w = (1, 16) if i == 0 else (4, 64)
        wts = _prep_conv(lp['conv'], F, T, nch, w)
        wts['Wih'] = lp['gru']['Wih']
        wts['Whh'] = lp['gru']['Whh']
        wts['bih'] = lp['gru']['bih'].reshape(1, -1)
        wts['bhh'] = lp['gru']['bhh'].reshape(1, -1)
        apply_bn = i > 0
        gamma = None if i == 0 else layers[i - 1]['bn_gamma']
        beta = None if i == 0 else layers[i - 1]['bn_beta']
        bt, cch, hbn = _tcA(i, apply_bn, nch, w, F, h, bn, gamma, beta,
                            wts['Amat'], wts['Bmat'], wts['bpre'])
        S1, S2, Mn, Mx = seg(bt, cch, nch, w)
        h, bn = _tcB(i, nch, w, F, H, True, hbn, cch, S1, S2, Mn, Mx,
                     counts_col, avg, wts)
    wts = _prep_conv(params['readout'], H, 1, 1, 64)
    bt, cch, hbn = _tcA(14, True, 1, 64, H, h, bn,
                        layers[-1]['bn_gamma'], layers[-1]['bn_beta'],
                        wts['Amat'], wts['Bmat'], wts['bpre'])
    S1, S2, Mn, Mx = seg(bt, cch, 1, 64)
    y, = _tcB(15, 1, 64, H, 1, False, hbn, cch, S1, S2, Mn, Mx,
              counts_col, avg, wts)
    return y[:N]
